# baseline (speedup 1.0000x reference)
import jax
import jax.numpy as jnp
from jax.experimental import pallas as pl
from jax.experimental.pallas import tpu as pltpu

N_EDGES = 3_200_000
N_NODES = 100_000
N_EDGE_ROWS = 400_000
GRID = 100
EBLK = N_EDGE_ROWS // GRID
NBLK = N_NODES // GRID


def _body(glob_ref, edges_ref, nodes_ref, WgT_ref, WeT_ref, WnT_ref, b_ref,
          out_ref, eacc, nacc):
    g = pl.program_id(0)

    @pl.when(g == 0)
    def _init():
        eacc[...] = jnp.zeros_like(eacc)
        nacc[...] = jnp.zeros_like(nacc)

    eacc[...] += jnp.sum(edges_ref[...], axis=0, keepdims=True)
    nacc[...] += jnp.sum(nodes_ref[...], axis=0, keepdims=True)

    @pl.when(g == GRID - 1)
    def _fin():
        e_row = eacc[...] * (1.0 / N_EDGES)
        n_row = nacc[...] * (1.0 / N_NODES)
        out = (jnp.dot(glob_ref[...], WgT_ref[...],
                       preferred_element_type=jnp.float32)
               + jnp.dot(e_row, WeT_ref[...],
                         preferred_element_type=jnp.float32)
               + jnp.dot(n_row, WnT_ref[...],
                         preferred_element_type=jnp.float32)
               + b_ref[...])
        out_ref[...] = out


def kernel(global_data, nodes_data, edges_data, W, b):
    edges2 = edges_data.reshape(N_EDGE_ROWS, 128)
    WT = W.T
    WgT = WT[:128]
    WeT = jnp.tile(WT[128:144], (8, 1))
    WnT = WT[144:]
    out = pl.pallas_call(
        _body,
        grid=(GRID,),
        in_specs=[
            pl.BlockSpec((1, 128), lambda g: (0, 0)),
            pl.BlockSpec((EBLK, 128), lambda g: (g, 0)),
            pl.BlockSpec((NBLK, 128), lambda g: (g, 0)),
            pl.BlockSpec((128, 128), lambda g: (0, 0)),
            pl.BlockSpec((128, 128), lambda g: (0, 0)),
            pl.BlockSpec((128, 128), lambda g: (0, 0)),
            pl.BlockSpec((1, 128), lambda g: (0, 0)),
        ],
        out_specs=pl.BlockSpec((1, 128), lambda g: (0, 0)),
        out_shape=jax.ShapeDtypeStruct((1, 128), jnp.float32),
        scratch_shapes=[
            pltpu.VMEM((1, 128), jnp.float32),
            pltpu.VMEM((1, 128), jnp.float32),
        ],
    )(global_data[None, :], edges2, nodes_data, WgT, WeT, WnT, b[None, :])
    return out[0]


# speedup vs baseline: 1.0010x; 1.0010x over previous
import jax
import jax.numpy as jnp
from jax.experimental import pallas as pl
from jax.experimental.pallas import tpu as pltpu

N_EDGES = 3_200_000
N_NODES = 100_000
N_EDGE_MAJ = 50_000
GRID = 100
EBLK = N_EDGE_MAJ // GRID
NBLK = N_NODES // GRID


def _body(glob_ref, edges_ref, nodes_ref, WgT_ref, WeT_ref, WnT_ref, b_ref,
          out_ref, eacc, nacc):
    g = pl.program_id(0)

    @pl.when(g == 0)
    def _init():
        eacc[...] = jnp.zeros_like(eacc)
        nacc[...] = jnp.zeros_like(nacc)

    eacc[...] += jnp.sum(edges_ref[...], axis=(0, 1))[None, :]
    nacc[...] += jnp.sum(nodes_ref[...], axis=0, keepdims=True)

    @pl.when(g == GRID - 1)
    def _fin():
        e_row = eacc[...] * (1.0 / N_EDGES)
        n_row = nacc[...] * (1.0 / N_NODES)
        out = (jnp.dot(glob_ref[...], WgT_ref[...],
                       preferred_element_type=jnp.float32)
               + jnp.dot(e_row, WeT_ref[...],
                         preferred_element_type=jnp.float32)
               + jnp.dot(n_row, WnT_ref[...],
                         preferred_element_type=jnp.float32)
               + b_ref[...])
        out_ref[...] = out


def kernel(global_data, nodes_data, edges_data, W, b):
    edges2 = edges_data.reshape(N_EDGE_MAJ, 8, 128)
    WT = W.T
    WgT = WT[:128]
    WeT = jnp.tile(WT[128:144], (8, 1))
    WnT = WT[144:]
    out = pl.pallas_call(
        _body,
        grid=(GRID,),
        in_specs=[
            pl.BlockSpec((1, 128), lambda g: (0, 0)),
            pl.BlockSpec((EBLK, 8, 128), lambda g: (g, 0, 0)),
            pl.BlockSpec((NBLK, 128), lambda g: (g, 0)),
            pl.BlockSpec((128, 128), lambda g: (0, 0)),
            pl.BlockSpec((128, 128), lambda g: (0, 0)),
            pl.BlockSpec((128, 128), lambda g: (0, 0)),
            pl.BlockSpec((1, 128), lambda g: (0, 0)),
        ],
        out_specs=pl.BlockSpec((1, 128), lambda g: (0, 0)),
        out_shape=jax.ShapeDtypeStruct((1, 128), jnp.float32),
        scratch_shapes=[
            pltpu.VMEM((1, 128), jnp.float32),
            pltpu.VMEM((1, 128), jnp.float32),
        ],
    )(global_data[None, :], edges2, nodes_data, WgT, WeT, WnT, b[None, :])
    return out[0]


# speedup vs baseline: 1.0284x; 1.0275x over previous
import jax
import jax.numpy as jnp
from jax import lax
from jax.experimental import pallas as pl
from jax.experimental.pallas import tpu as pltpu
from jax.experimental.pallas import tpu_sc as plsc

N_EDGES = 3_200_000
N_NODES = 100_000
D_EDGE = 16
D_FEAT = 128

NW = 32
E_PER_W = N_EDGES // NW
N_PER_W = 3120
N_TAIL = N_NODES - NW * N_PER_W
ECHUNK = 2000
NCHUNK = 240
EUNROLL = 8


def _sc_body(edges_hbm, nodes_hbm, pe_hbm, pn_hbm,
             ebuf, nbuf, pbuf, sem):
    wid = lax.axis_index("s") * 2 + lax.axis_index("c")
    ebase = wid * E_PER_W
    nbase = wid * N_PER_W

    def echunk_body(ci, eacc):
        pltpu.sync_copy(edges_hbm.at[pl.ds(ebase + ci * ECHUNK, ECHUNK)],
                        ebuf)

        def erow_body(i, acc):
            return tuple(
                acc[j] + ebuf[i * EUNROLL + j, :] for j in range(EUNROLL)
            )

        return lax.fori_loop(0, ECHUNK // EUNROLL, erow_body, eacc)

    ezero = tuple(jnp.zeros((16,), jnp.float32) for _ in range(EUNROLL))
    eacc = lax.fori_loop(0, E_PER_W // ECHUNK, echunk_body, ezero)
    esum = eacc[0]
    for j in range(1, EUNROLL):
        esum = esum + eacc[j]
    pbuf[0, pl.ds(0, 16)] = esum

    def nchunk_body(ci, nacc):
        pltpu.sync_copy(nodes_hbm.at[pl.ds(nbase + ci * NCHUNK, NCHUNK)],
                        nbuf)

        def nrow_body(i, acc):
            return tuple(
                acc[k] + nbuf[i, pl.ds(16 * k, 16)] for k in range(8)
            )

        return lax.fori_loop(0, NCHUNK, nrow_body, nacc)

    nzero = tuple(jnp.zeros((16,), jnp.float32) for _ in range(8))
    nacc = lax.fori_loop(0, N_PER_W // NCHUNK, nchunk_body, nzero)
    for k in range(8):
        pbuf[1, pl.ds(16 * k, 16)] = nacc[k]

    pltpu.sync_copy(pbuf.at[0, pl.ds(0, 16)], pe_hbm.at[wid])
    pltpu.sync_copy(pbuf.at[1], pn_hbm.at[wid])


def _sc_sums(edges_data, nodes_data):
    mesh = plsc.VectorSubcoreMesh(core_axis_name="c", subcore_axis_name="s")
    return pl.kernel(
        _sc_body,
        mesh=mesh,
        compiler_params=pltpu.CompilerParams(use_tc_tiling_on_sc=False),
        out_type=[
            jax.ShapeDtypeStruct((NW, D_EDGE), jnp.float32),
            jax.ShapeDtypeStruct((NW, D_FEAT), jnp.float32),
        ],
        scratch_types=[
            pltpu.VMEM((ECHUNK, D_EDGE), jnp.float32),
            pltpu.VMEM((NCHUNK, D_FEAT), jnp.float32),
            pltpu.VMEM((2, D_FEAT), jnp.float32),
            pltpu.SemaphoreType.DMA,
        ],
    )(edges_data, nodes_data)


def _tc_body(glob_ref, pe_ref, pn_ref, ntail_ref, WgT_ref, WeT_ref, WnT_ref,
             b_ref, out_ref):
    e_row = jnp.sum(pe_ref[...], axis=0, keepdims=True) * (1.0 / N_EDGES)
    n_row = (jnp.sum(pn_ref[...], axis=0, keepdims=True)
             + jnp.sum(ntail_ref[...], axis=0, keepdims=True)) \
        * (1.0 / N_NODES)
    out_ref[...] = (
        jnp.dot(glob_ref[...], WgT_ref[...],
                preferred_element_type=jnp.float32)
        + jnp.dot(e_row, WeT_ref[...], preferred_element_type=jnp.float32)
        + jnp.dot(n_row, WnT_ref[...], preferred_element_type=jnp.float32)
        + b_ref[...])


def kernel(global_data, nodes_data, edges_data, W, b):
    pe, pn = _sc_sums(edges_data, nodes_data)
    WT = W.T
    out = pl.pallas_call(
        _tc_body,
        out_shape=jax.ShapeDtypeStruct((1, 128), jnp.float32),
    )(global_data[None, :], pe, pn, nodes_data[NW * N_PER_W:],
      WT[:128], WT[128:144], WT[144:], b[None, :])
    return out[0]


# speedup vs baseline: 11.6121x; 11.2909x over previous
import jax
import jax.numpy as jnp
from jax import lax
from jax.experimental import pallas as pl
from jax.experimental.pallas import tpu as pltpu
from jax.experimental.pallas import tpu_sc as plsc

N_EDGES = 3_200_000
N_NODES = 100_000
D_EDGE = 16
D_FEAT = 128

NW = 32
CHUNK = 3200
NCHUNKS = N_EDGES // CHUNK
MAIN = 30
PER_W = 31
TAIL_W = NCHUNKS - NW * PER_W

TC_GRID = 100
NBLK = N_NODES // TC_GRID


def _sc_body(edges_hbm, pe_hbm, buf0, buf1, pbuf, sem0, sem1):
    wid = lax.axis_index("s") * 2 + lax.axis_index("c")
    base = wid * PER_W

    def start(ci, buf, sem):
        off = jnp.minimum(base + ci, NCHUNKS - 1) * CHUNK
        pltpu.async_copy(edges_hbm.at[:, pl.ds(off, CHUNK)], buf, sem)

    def drain(buf, sem):
        pltpu.make_async_copy(edges_hbm.at[:, pl.ds(0, CHUNK)], buf,
                              sem).wait()

    def accum(buf, acc):
        def body(k, a):
            return tuple(
                a[r] + buf[r, pl.ds(k * 16, 16)] for r in range(16)
            )
        return lax.fori_loop(0, CHUNK // 16, body, acc)

    start(0, buf0, sem0)

    def pair_body(i, acc):
        start(2 * i + 1, buf1, sem1)
        drain(buf0, sem0)
        acc = accum(buf0, acc)
        start(2 * i + 2, buf0, sem0)
        drain(buf1, sem1)
        return accum(buf1, acc)

    zero = tuple(jnp.zeros((16,), jnp.float32) for _ in range(16))
    acc = lax.fori_loop(0, MAIN // 2, pair_body, zero)
    drain(buf0, sem0)
    acc = accum(buf0, acc)

    toff = jnp.minimum(NW * PER_W + wid, NCHUNKS - 1) * CHUNK
    pltpu.async_copy(edges_hbm.at[:, pl.ds(toff, CHUNK)], buf0, sem0).wait()
    tw = jnp.where(wid < TAIL_W, jnp.float32(1.0), jnp.float32(0.0))

    def tail_body(k, a):
        return tuple(
            a[r] + tw * buf0[r, pl.ds(k * 16, 16)] for r in range(16)
        )

    acc = lax.fori_loop(0, CHUNK // 16, tail_body, acc)

    for r in range(16):
        pbuf[r, :] = acc[r]
    pltpu.sync_copy(pbuf, pe_hbm.at[pl.ds(wid * 16, 16)])


def _sc_edge_sums(edges_t):
    mesh = plsc.VectorSubcoreMesh(core_axis_name="c", subcore_axis_name="s")
    return pl.kernel(
        _sc_body,
        mesh=mesh,
        out_type=jax.ShapeDtypeStruct((NW * 16, D_EDGE), jnp.float32),
        scratch_types=[
            pltpu.VMEM((D_EDGE, CHUNK), jnp.float32),
            pltpu.VMEM((D_EDGE, CHUNK), jnp.float32),
            pltpu.VMEM((16, D_EDGE), jnp.float32),
            pltpu.SemaphoreType.DMA,
            pltpu.SemaphoreType.DMA,
        ],
    )(edges_t)


def _tc_nodes_body(nodes_ref, nsum_ref, nacc):
    g = pl.program_id(0)

    @pl.when(g == 0)
    def _init():
        nacc[...] = jnp.zeros_like(nacc)

    nacc[...] += jnp.sum(nodes_ref[...], axis=0, keepdims=True)

    @pl.when(g == TC_GRID - 1)
    def _fin():
        nsum_ref[...] = nacc[...]


def _tc_node_sum(nodes_data):
    return pl.pallas_call(
        _tc_nodes_body,
        grid=(TC_GRID,),
        in_specs=[pl.BlockSpec((NBLK, 128), lambda g: (g, 0))],
        out_specs=pl.BlockSpec((1, 128), lambda g: (0, 0)),
        out_shape=jax.ShapeDtypeStruct((1, 128), jnp.float32),
        scratch_shapes=[pltpu.VMEM((1, 128), jnp.float32)],
    )(nodes_data)


def _tc_fin_body(glob_ref, pe_ref, nsum_ref, WgT_ref, WeRep_ref, WnT_ref,
                 b_ref, out_ref):
    rowsum = jnp.dot(pe_ref[...], jnp.ones((16, 1), jnp.float32),
                     preferred_element_type=jnp.float32)
    e_out = lax.dot_general(
        rowsum, WeRep_ref[...], (((0,), (0,)), ((), ())),
        preferred_element_type=jnp.float32) * (1.0 / N_EDGES)
    n_row = nsum_ref[...] * (1.0 / N_NODES)
    out_ref[...] = (
        jnp.dot(glob_ref[...], WgT_ref[...],
                preferred_element_type=jnp.float32)
        + e_out
        + jnp.dot(n_row, WnT_ref[...], preferred_element_type=jnp.float32)
        + b_ref[...])


def kernel(global_data, nodes_data, edges_data, W, b):
    pe = _sc_edge_sums(edges_data.T)
    nsum = _tc_node_sum(nodes_data)
    WT = W.T
    WeRep = jnp.tile(WT[128:144], (NW, 1))
    out = pl.pallas_call(
        _tc_fin_body,
        out_shape=jax.ShapeDtypeStruct((1, 128), jnp.float32),
    )(global_data[None, :], pe, nsum, WT[:128], WeRep, WT[144:],
      b[None, :])
    return out[0]
